# Initial kernel scaffold; baseline (speedup 1.0000x reference)
import jax
import jax.numpy as jnp
from jax.experimental import pallas as pl


def kernel(in_tensor, line_z, line_y, line_x):
    raise NotImplementedError("write your pallas kernel here")


# speedup vs baseline: 21.7034x; 21.7034x over previous
import functools

import jax
import jax.numpy as jnp
from jax import lax
from jax.experimental import pallas as pl
from jax.experimental.pallas import tpu as pltpu, tpu_sc as plsc

C = 96
R = 256
N = 524288

NUM_WORKERS = 32
PTS_PER_WORKER = N // NUM_WORKERS
CHUNK = 2048
NUM_CHUNKS = PTS_PER_WORKER // CHUNK
GROUPS = CHUNK // 16

_TABLE_WORDS = 3 * C * R


def _decode_body(cx_hbm, cy_hbm, cz_hbm, table_hbm, out_hbm,
                 table_v, cx_v, cy_v, cz_v, out_v):
    wid = lax.axis_index("s") * 2 + lax.axis_index("c")
    base = wid * PTS_PER_WORKER

    pltpu.sync_copy(table_hbm, table_v)

    def axis_prep(coord):
        pos = (coord + 1.0) * ((R - 1) * 0.5)
        i0 = jnp.clip(pos.astype(jnp.int32), 0, R - 2)
        w = pos - i0.astype(jnp.float32)
        return i0, w

    def do_chunk(k, _):
        off = base + k * CHUNK
        pltpu.sync_copy(cx_hbm.at[pl.ds(off, CHUNK)], cx_v)
        pltpu.sync_copy(cy_hbm.at[pl.ds(off, CHUNK)], cy_v)
        pltpu.sync_copy(cz_hbm.at[pl.ds(off, CHUNK)], cz_v)

        def do_group(g, _):
            s = pl.ds(g * 16, 16)
            ix0, wx = axis_prep(cx_v[s])
            iy0, wy = axis_prep(cy_v[s])
            iz0, wz = axis_prep(cz_v[s])

            accs = [jnp.zeros((16,), jnp.float32) for _ in range(4)]
            for c in range(C):
                bx = c * R
                by = (C + c) * R
                bz = (2 * C + c) * R
                fx0 = plsc.load_gather(table_v, [ix0 + bx])
                fx1 = plsc.load_gather(table_v, [ix0 + (bx + 1)])
                fy0 = plsc.load_gather(table_v, [iy0 + by])
                fy1 = plsc.load_gather(table_v, [iy0 + (by + 1)])
                fz0 = plsc.load_gather(table_v, [iz0 + bz])
                fz1 = plsc.load_gather(table_v, [iz0 + (bz + 1)])
                fx = fx0 + wx * (fx1 - fx0)
                fy = fy0 + wy * (fy1 - fy0)
                fz = fz0 + wz * (fz1 - fz0)
                accs[c % 4] = accs[c % 4] + fx * fy * fz
            out_v[s] = (accs[0] + accs[1]) + (accs[2] + accs[3])
            return 0

        lax.fori_loop(0, GROUPS, do_group, 0)
        pltpu.sync_copy(out_v, out_hbm.at[pl.ds(off, CHUNK)])
        return 0

    lax.fori_loop(0, NUM_CHUNKS, do_chunk, 0)


@jax.jit
def kernel(in_tensor, line_z, line_y, line_x):
    mesh = plsc.VectorSubcoreMesh(core_axis_name="c", subcore_axis_name="s")
    table = jnp.concatenate(
        [line_x.reshape(-1), line_y.reshape(-1), line_z.reshape(-1)])
    run = functools.partial(
        pl.kernel,
        mesh=mesh,
        compiler_params=pltpu.CompilerParams(needs_layout_passes=False),
        out_type=jax.ShapeDtypeStruct((N,), jnp.float32),
        scratch_types=[
            pltpu.VMEM((_TABLE_WORDS,), jnp.float32),
            pltpu.VMEM((CHUNK,), jnp.float32),
            pltpu.VMEM((CHUNK,), jnp.float32),
            pltpu.VMEM((CHUNK,), jnp.float32),
            pltpu.VMEM((CHUNK,), jnp.float32),
        ],
    )(_decode_body)
    return run(in_tensor[:, 0], in_tensor[:, 1], in_tensor[:, 2], table)


# speedup vs baseline: 39.1530x; 1.8040x over previous
import functools

import jax
import jax.numpy as jnp
from jax import lax
from jax.experimental import pallas as pl
from jax.experimental.pallas import tpu as pltpu, tpu_sc as plsc

C = 96
R = 256
N = 524288

BZ = 8

NUM_WORKERS = 32
PTS_PER_WORKER = N // NUM_WORKERS
CHUNK = 2048
NUM_CHUNKS = PTS_PER_WORKER // CHUNK
GROUPS = CHUNK // 16
IDX_PER_CHUNK = 4 * CHUNK
DMA_ELS = 128
NUM_DMAS = IDX_PER_CHUNK // DMA_ELS


def _build_volume_body(lz_ref, ly_ref, lx0_ref, lx1_ref, out_ref):
    gstep = pl.program_id(0)
    lz = lz_ref[...]
    ly = ly_ref[...]
    lx0 = lx0_ref[...]
    lx1 = lx1_ref[...]
    lane = lax.broadcasted_iota(jnp.int32, (1, R), 1)
    for b in range(BZ):
        onehot = (lane == gstep * BZ + b).astype(jnp.float32)
        wcol = jnp.sum(lz * onehot, axis=1, keepdims=True)
        w = wcol * ly
        dn = (((0,), (0,)), ((), ()))
        g0 = lax.dot_general(w, lx0, dn, preferred_element_type=jnp.float32)
        g1 = lax.dot_general(w, lx1, dn, preferred_element_type=jnp.float32)
        u0 = lax.bitcast_convert_type(g0, jnp.uint32)
        u1 = lax.bitcast_convert_type(g1, jnp.uint32)
        word = ((u0 + 0x8000) >> 16) | ((u1 + 0x8000) & jnp.uint32(0xFFFF0000))
        out_ref[b] = word


def _build_volume(line_z, line_y, line_x):
    lx1 = jnp.concatenate([line_x[:, 1:], line_x[:, -1:]], axis=1)
    grid = R // BZ
    return pl.pallas_call(
        _build_volume_body,
        grid=(grid,),
        in_specs=[
            pl.BlockSpec((C, R), lambda g: (0, 0)),
            pl.BlockSpec((C, R), lambda g: (0, 0)),
            pl.BlockSpec((C, R), lambda g: (0, 0)),
            pl.BlockSpec((C, R), lambda g: (0, 0)),
        ],
        out_specs=pl.BlockSpec((BZ, R, R), lambda g: (g, 0, 0)),
        out_shape=jax.ShapeDtypeStruct((R, R, R), jnp.uint32),
    )(line_z, line_y, line_x, lx1)


def _sample_body(cx_hbm, cy_hbm, cz_hbm, vol_hbm, out_hbm,
                 cx_v, cy_v, cz_v, wx_v, wy_v, wz_v,
                 idx_v, val_v, out_v, sem):
    wid = lax.axis_index("s") * 2 + lax.axis_index("c")
    base = wid * PTS_PER_WORKER

    def axis_prep(coord):
        pos = (coord + 1.0) * ((R - 1) * 0.5)
        i0 = jnp.clip(pos.astype(jnp.int32), 0, R - 2)
        w = pos - i0.astype(jnp.float32)
        return i0, w

    def unpack_lo(v):
        return plsc.bitcast(v << 16, jnp.float32)

    def unpack_hi(v):
        return plsc.bitcast(v & jnp.int32(-65536), jnp.float32)

    def do_chunk(k, _):
        off = base + k * CHUNK
        pltpu.sync_copy(cx_hbm.at[pl.ds(off, CHUNK)], cx_v)
        pltpu.sync_copy(cy_hbm.at[pl.ds(off, CHUNK)], cy_v)
        pltpu.sync_copy(cz_hbm.at[pl.ds(off, CHUNK)], cz_v)

        def pass_a(g, _):
            s = pl.ds(g * 16, 16)
            ix0, wx = axis_prep(cx_v[s])
            iy0, wy = axis_prep(cy_v[s])
            iz0, wz = axis_prep(cz_v[s])
            wx_v[s] = wx
            wy_v[s] = wy
            wz_v[s] = wz
            base_w = (iz0 << 16) + (iy0 << 8) + ix0
            idx_v[pl.ds(0 * CHUNK + g * 16, 16)] = base_w
            idx_v[pl.ds(1 * CHUNK + g * 16, 16)] = base_w + 256
            idx_v[pl.ds(2 * CHUNK + g * 16, 16)] = base_w + 65536
            idx_v[pl.ds(3 * CHUNK + g * 16, 16)] = base_w + 65536 + 256
            return 0

        lax.fori_loop(0, GROUPS, pass_a, 0)

        copies = []
        for t in range(NUM_DMAS):
            sl = pl.ds(t * DMA_ELS, DMA_ELS)
            copies.append(
                pltpu.async_copy(vol_hbm.at[idx_v.at[sl]], val_v.at[sl], sem))
        for cp in copies:
            cp.wait()

        def pass_b(g, _):
            s = pl.ds(g * 16, 16)
            v00 = val_v[pl.ds(0 * CHUNK + g * 16, 16)]
            v01 = val_v[pl.ds(1 * CHUNK + g * 16, 16)]
            v10 = val_v[pl.ds(2 * CHUNK + g * 16, 16)]
            v11 = val_v[pl.ds(3 * CHUNK + g * 16, 16)]
            wx = wx_v[s]
            wy = wy_v[s]
            wz = wz_v[s]

            def xl(v):
                f0 = unpack_lo(v)
                f1 = unpack_hi(v)
                return f0 + wx * (f1 - f0)

            q00 = xl(v00)
            q01 = xl(v01)
            q10 = xl(v10)
            q11 = xl(v11)
            r0 = q00 + wy * (q01 - q00)
            r1 = q10 + wy * (q11 - q10)
            out_v[s] = r0 + wz * (r1 - r0)
            return 0

        lax.fori_loop(0, GROUPS, pass_b, 0)
        pltpu.sync_copy(out_v, out_hbm.at[pl.ds(off, CHUNK)])
        return 0

    lax.fori_loop(0, NUM_CHUNKS, do_chunk, 0)


@jax.jit
def kernel(in_tensor, line_z, line_y, line_x):
    vol = _build_volume(line_z, line_y, line_x).reshape(-1).view(jnp.int32)
    mesh = plsc.VectorSubcoreMesh(core_axis_name="c", subcore_axis_name="s")
    run = functools.partial(
        pl.kernel,
        mesh=mesh,
        compiler_params=pltpu.CompilerParams(needs_layout_passes=False),
        out_type=jax.ShapeDtypeStruct((N,), jnp.float32),
        scratch_types=[
            pltpu.VMEM((CHUNK,), jnp.float32),
            pltpu.VMEM((CHUNK,), jnp.float32),
            pltpu.VMEM((CHUNK,), jnp.float32),
            pltpu.VMEM((CHUNK,), jnp.float32),
            pltpu.VMEM((CHUNK,), jnp.float32),
            pltpu.VMEM((CHUNK,), jnp.float32),
            pltpu.VMEM((IDX_PER_CHUNK,), jnp.int32),
            pltpu.VMEM((IDX_PER_CHUNK,), jnp.int32),
            pltpu.VMEM((CHUNK,), jnp.float32),
            pltpu.SemaphoreType.DMA,
        ],
    )(_sample_body)
    return run(in_tensor[:, 0], in_tensor[:, 1], in_tensor[:, 2], vol)


# speedup vs baseline: 40.5915x; 1.0367x over previous
import functools

import jax
import jax.numpy as jnp
from jax import lax
from jax.experimental import pallas as pl
from jax.experimental.pallas import tpu as pltpu, tpu_sc as plsc

C = 96
R = 256
N = 524288

BZ = 8

NUM_WORKERS = 32
PTS_PER_WORKER = N // NUM_WORKERS
CHUNK = 4096
NUM_CHUNKS = PTS_PER_WORKER // CHUNK
GROUPS = CHUNK // 16
IDX_PER_CHUNK = 4 * CHUNK
IDX_ROWS = IDX_PER_CHUNK // 128
ROWS_PER_CORNER = CHUNK // 128


def _build_volume_body(lz_ref, ly_ref, lx0_ref, lx1_ref, out_ref):
    gstep = pl.program_id(0)
    lz = lz_ref[...]
    ly = ly_ref[...]
    lx0 = lx0_ref[...]
    lx1 = lx1_ref[...]
    lane = lax.broadcasted_iota(jnp.int32, (1, R), 1)
    for b in range(BZ):
        onehot = (lane == gstep * BZ + b).astype(jnp.float32)
        wcol = jnp.sum(lz * onehot, axis=1, keepdims=True)
        w = wcol * ly
        dn = (((0,), (0,)), ((), ()))
        g0 = lax.dot_general(w, lx0, dn, preferred_element_type=jnp.float32)
        g1 = lax.dot_general(w, lx1, dn, preferred_element_type=jnp.float32)
        u0 = lax.bitcast_convert_type(g0, jnp.uint32)
        u1 = lax.bitcast_convert_type(g1, jnp.uint32)
        word = ((u0 + 0x8000) >> 16) | ((u1 + 0x8000) & jnp.uint32(0xFFFF0000))
        out_ref[b] = word


def _build_volume(line_z, line_y, line_x):
    lx1 = jnp.concatenate([line_x[:, 1:], line_x[:, -1:]], axis=1)
    grid = R // BZ
    return pl.pallas_call(
        _build_volume_body,
        grid=(grid,),
        in_specs=[
            pl.BlockSpec((C, R), lambda g: (0, 0)),
            pl.BlockSpec((C, R), lambda g: (0, 0)),
            pl.BlockSpec((C, R), lambda g: (0, 0)),
            pl.BlockSpec((C, R), lambda g: (0, 0)),
        ],
        out_specs=pl.BlockSpec((BZ, R, R), lambda g: (g, 0, 0)),
        out_shape=jax.ShapeDtypeStruct((R, R, R), jnp.uint32),
    )(line_z, line_y, line_x, lx1)


def _sample_body(cx_hbm, cy_hbm, cz_hbm, vol_hbm, out_hbm,
                 cx_v, cy_v, cz_v, wx_v, wy_v, wz_v,
                 idx_v, val_v, out_v, sem):
    wid = lax.axis_index("s") * 2 + lax.axis_index("c")
    base = wid * PTS_PER_WORKER

    def axis_prep(coord):
        pos = (coord + 1.0) * ((R - 1) * 0.5)
        i0 = jnp.clip(pos.astype(jnp.int32), 0, R - 2)
        w = pos - i0.astype(jnp.float32)
        return i0, w

    def unpack_lo(v):
        return plsc.bitcast(v << 16, jnp.float32)

    def unpack_hi(v):
        return plsc.bitcast(v & jnp.int32(-65536), jnp.float32)

    def do_chunk(k, _):
        off = base + k * CHUNK
        pltpu.sync_copy(cx_hbm.at[pl.ds(off, CHUNK)], cx_v)
        pltpu.sync_copy(cy_hbm.at[pl.ds(off, CHUNK)], cy_v)
        pltpu.sync_copy(cz_hbm.at[pl.ds(off, CHUNK)], cz_v)

        def pass_a(g, _):
            s = pl.ds(g * 16, 16)
            ix0, wx = axis_prep(cx_v[s])
            iy0, wy = axis_prep(cy_v[s])
            iz0, wz = axis_prep(cz_v[s])
            wx_v[s] = wx
            wy_v[s] = wy
            wz_v[s] = wz
            base_w = (iz0 << 16) + (iy0 << 8) + ix0
            idx_v[pl.ds(0 * CHUNK + g * 16, 16)] = base_w
            idx_v[pl.ds(1 * CHUNK + g * 16, 16)] = base_w + 256
            idx_v[pl.ds(2 * CHUNK + g * 16, 16)] = base_w + 65536
            idx_v[pl.ds(3 * CHUNK + g * 16, 16)] = base_w + 65536 + 256
            return 0

        lax.fori_loop(0, GROUPS, pass_a, 0)

        pltpu.async_copy(vol_hbm.at[idx_v], val_v, sem).wait()

        def pass_b(g, _):
            s = pl.ds(g * 16, 16)
            v00 = val_v[pl.ds(0 * CHUNK + g * 16, 16)]
            v01 = val_v[pl.ds(1 * CHUNK + g * 16, 16)]
            v10 = val_v[pl.ds(2 * CHUNK + g * 16, 16)]
            v11 = val_v[pl.ds(3 * CHUNK + g * 16, 16)]
            wx = wx_v[s]
            wy = wy_v[s]
            wz = wz_v[s]

            def xl(v):
                f0 = unpack_lo(v)
                f1 = unpack_hi(v)
                return f0 + wx * (f1 - f0)

            q00 = xl(v00)
            q01 = xl(v01)
            q10 = xl(v10)
            q11 = xl(v11)
            r0 = q00 + wy * (q01 - q00)
            r1 = q10 + wy * (q11 - q10)
            out_v[s] = r0 + wz * (r1 - r0)
            return 0

        lax.fori_loop(0, GROUPS, pass_b, 0)
        pltpu.sync_copy(out_v, out_hbm.at[pl.ds(off, CHUNK)])
        return 0

    lax.fori_loop(0, NUM_CHUNKS, do_chunk, 0)


@jax.jit
def kernel(in_tensor, line_z, line_y, line_x):
    vol = _build_volume(line_z, line_y, line_x).reshape(-1).view(jnp.int32)
    mesh = plsc.VectorSubcoreMesh(core_axis_name="c", subcore_axis_name="s")
    run = functools.partial(
        pl.kernel,
        mesh=mesh,
        compiler_params=pltpu.CompilerParams(needs_layout_passes=False),
        out_type=jax.ShapeDtypeStruct((N,), jnp.float32),
        scratch_types=[
            pltpu.VMEM((CHUNK,), jnp.float32),
            pltpu.VMEM((CHUNK,), jnp.float32),
            pltpu.VMEM((CHUNK,), jnp.float32),
            pltpu.VMEM((CHUNK,), jnp.float32),
            pltpu.VMEM((CHUNK,), jnp.float32),
            pltpu.VMEM((CHUNK,), jnp.float32),
            pltpu.VMEM((IDX_PER_CHUNK,), jnp.int32),
            pltpu.VMEM((IDX_PER_CHUNK,), jnp.int32),
            pltpu.VMEM((CHUNK,), jnp.float32),
            pltpu.SemaphoreType.DMA,
        ],
    )(_sample_body)
    return run(in_tensor[:, 0], in_tensor[:, 1], in_tensor[:, 2], vol)


# speedup vs baseline: 48.1671x; 1.1866x over previous
import functools

import jax
import jax.numpy as jnp
from jax import lax
from jax.experimental import pallas as pl
from jax.experimental.pallas import tpu as pltpu, tpu_sc as plsc

C = 96
R = 256
N = 524288

BZ = 8

NUM_WORKERS = 32
PTS_PER_WORKER = N // NUM_WORKERS
CHUNK = 4096
NUM_CHUNKS = PTS_PER_WORKER // CHUNK
GROUPS = CHUNK // 16
IDX_PER_CHUNK = 4 * CHUNK
IDX_ROWS = IDX_PER_CHUNK // 128
ROWS_PER_CORNER = CHUNK // 128


def _build_volume_body(lz_ref, ly_ref, lx0_ref, lx1_ref, out_ref):
    gstep = pl.program_id(0)
    lz = lz_ref[...]
    ly = ly_ref[...]
    lx0 = lx0_ref[...]
    lx1 = lx1_ref[...]
    lane = lax.broadcasted_iota(jnp.int32, (1, R), 1)
    for b in range(BZ):
        onehot = (lane == gstep * BZ + b).astype(jnp.float32)
        wcol = jnp.sum(lz * onehot, axis=1, keepdims=True)
        w = wcol * ly
        dn = (((0,), (0,)), ((), ()))
        g0 = lax.dot_general(w, lx0, dn, preferred_element_type=jnp.float32)
        g1 = lax.dot_general(w, lx1, dn, preferred_element_type=jnp.float32)
        u0 = lax.bitcast_convert_type(g0, jnp.uint32)
        u1 = lax.bitcast_convert_type(g1, jnp.uint32)
        word = ((u0 + 0x8000) >> 16) | ((u1 + 0x8000) & jnp.uint32(0xFFFF0000))
        out_ref[pl.ds((b * 2 + 0) * 256, 256), :] = word[:, 0:128]
        out_ref[pl.ds((b * 2 + 1) * 256, 256), :] = word[:, 128:256]


def _build_volume(line_z, line_y, line_x):
    lx1 = jnp.concatenate([line_x[:, 1:], line_x[:, -1:]], axis=1)
    grid = R // BZ
    return pl.pallas_call(
        _build_volume_body,
        grid=(grid,),
        in_specs=[
            pl.BlockSpec((C, R), lambda g: (0, 0)),
            pl.BlockSpec((C, R), lambda g: (0, 0)),
            pl.BlockSpec((C, R), lambda g: (0, 0)),
            pl.BlockSpec((C, R), lambda g: (0, 0)),
        ],
        out_specs=pl.BlockSpec((BZ * 2 * R, 128), lambda g: (g, 0)),
        out_shape=jax.ShapeDtypeStruct((2 * R * R, 128), jnp.uint32),
    )(line_z, line_y, line_x, lx1)


def _sample_body(cx_hbm, cy_hbm, cz_hbm, vol_hbm, out_hbm,
                 cx_v, cy_v, cz_v, wx_v, wy_v, wz_v,
                 idx_v, val_v, out_v, sem):
    wid = lax.axis_index("s") * 2 + lax.axis_index("c")
    base = wid * PTS_PER_WORKER

    def axis_prep(coord):
        pos = (coord + 1.0) * ((R - 1) * 0.5)
        i0 = jnp.clip(pos.astype(jnp.int32), 0, R - 2)
        w = pos - i0.astype(jnp.float32)
        return i0, w

    def unpack_lo(v):
        return plsc.bitcast(v << 16, jnp.float32)

    def unpack_hi(v):
        return plsc.bitcast(v & jnp.int32(-65536), jnp.float32)

    def do_chunk(k, _):
        off = base + k * CHUNK
        pltpu.sync_copy(cx_hbm.at[pl.ds(off, CHUNK)], cx_v)
        pltpu.sync_copy(cy_hbm.at[pl.ds(off, CHUNK)], cy_v)
        pltpu.sync_copy(cz_hbm.at[pl.ds(off, CHUNK)], cz_v)

        def pass_a(g, _):
            s = pl.ds(g * 16, 16)
            ix0, wx = axis_prep(cx_v[s])
            iy0, wy = axis_prep(cy_v[s])
            iz0, wz = axis_prep(cz_v[s])
            wx_v[s] = wx
            wy_v[s] = wy
            wz_v[s] = wz
            base_w = (iz0 << 16) + ((ix0 >> 7) << 15) + (iy0 << 7) + (ix0 & 127)
            idx_v[pl.ds(0 * CHUNK + g * 16, 16)] = base_w
            idx_v[pl.ds(1 * CHUNK + g * 16, 16)] = base_w + 128
            idx_v[pl.ds(2 * CHUNK + g * 16, 16)] = base_w + 65536
            idx_v[pl.ds(3 * CHUNK + g * 16, 16)] = base_w + 65536 + 128
            return 0

        lax.fori_loop(0, GROUPS, pass_a, 0)

        pltpu.async_copy(vol_hbm.at[idx_v], val_v, sem).wait()

        def pass_b(g, _):
            s = pl.ds(g * 16, 16)
            v00 = val_v[pl.ds(0 * CHUNK + g * 16, 16)]
            v01 = val_v[pl.ds(1 * CHUNK + g * 16, 16)]
            v10 = val_v[pl.ds(2 * CHUNK + g * 16, 16)]
            v11 = val_v[pl.ds(3 * CHUNK + g * 16, 16)]
            wx = wx_v[s]
            wy = wy_v[s]
            wz = wz_v[s]

            def xl(v):
                f0 = unpack_lo(v)
                f1 = unpack_hi(v)
                return f0 + wx * (f1 - f0)

            q00 = xl(v00)
            q01 = xl(v01)
            q10 = xl(v10)
            q11 = xl(v11)
            r0 = q00 + wy * (q01 - q00)
            r1 = q10 + wy * (q11 - q10)
            out_v[s] = r0 + wz * (r1 - r0)
            return 0

        lax.fori_loop(0, GROUPS, pass_b, 0)
        pltpu.sync_copy(out_v, out_hbm.at[pl.ds(off, CHUNK)])
        return 0

    lax.fori_loop(0, NUM_CHUNKS, do_chunk, 0)


@jax.jit
def kernel(in_tensor, line_z, line_y, line_x):
    vol = _build_volume(line_z, line_y, line_x).reshape(-1).view(jnp.int32)
    mesh = plsc.VectorSubcoreMesh(core_axis_name="c", subcore_axis_name="s")
    run = functools.partial(
        pl.kernel,
        mesh=mesh,
        compiler_params=pltpu.CompilerParams(needs_layout_passes=False),
        out_type=jax.ShapeDtypeStruct((N,), jnp.float32),
        scratch_types=[
            pltpu.VMEM((CHUNK,), jnp.float32),
            pltpu.VMEM((CHUNK,), jnp.float32),
            pltpu.VMEM((CHUNK,), jnp.float32),
            pltpu.VMEM((CHUNK,), jnp.float32),
            pltpu.VMEM((CHUNK,), jnp.float32),
            pltpu.VMEM((CHUNK,), jnp.float32),
            pltpu.VMEM((IDX_PER_CHUNK,), jnp.int32),
            pltpu.VMEM((IDX_PER_CHUNK,), jnp.int32),
            pltpu.VMEM((CHUNK,), jnp.float32),
            pltpu.SemaphoreType.DMA,
        ],
    )(_sample_body)
    return run(in_tensor[:, 0], in_tensor[:, 1], in_tensor[:, 2], vol)


# speedup vs baseline: 51.8482x; 1.0764x over previous
import functools

import jax
import jax.numpy as jnp
from jax import lax
from jax.experimental import pallas as pl
from jax.experimental.pallas import tpu as pltpu, tpu_sc as plsc

C = 96
R = 256
N = 524288

BZ = 8

NUM_WORKERS = 32
PTS_PER_WORKER = N // NUM_WORKERS
CHUNK = 2048
NUM_CHUNKS = PTS_PER_WORKER // CHUNK
GROUPS = CHUNK // 16
IDX_PER_CHUNK = 4 * CHUNK


def _build_volume_body(lz_ref, ly_ref, lx0_ref, lx1_ref, out_ref):
    gstep = pl.program_id(0)
    lz = lz_ref[...]
    ly = ly_ref[...]
    lx0 = lx0_ref[...]
    lx1 = lx1_ref[...]
    lane = lax.broadcasted_iota(jnp.int32, (1, R), 1)
    for b in range(BZ):
        onehot = (lane == gstep * BZ + b).astype(jnp.float32)
        wcol = jnp.sum(lz * onehot, axis=1, keepdims=True)
        w = wcol * ly
        dn = (((0,), (0,)), ((), ()))
        g0 = lax.dot_general(w, lx0, dn, preferred_element_type=jnp.float32)
        g1 = lax.dot_general(w, lx1, dn, preferred_element_type=jnp.float32)
        u0 = lax.bitcast_convert_type(g0, jnp.uint32)
        u1 = lax.bitcast_convert_type(g1, jnp.uint32)
        word = ((u0 + 0x8000) >> 16) | ((u1 + 0x8000) & jnp.uint32(0xFFFF0000))
        out_ref[pl.ds((b * 2 + 0) * 256, 256), :] = word[:, 0:128]
        out_ref[pl.ds((b * 2 + 1) * 256, 256), :] = word[:, 128:256]


def _build_volume(line_z, line_y, line_x):
    lx1 = jnp.concatenate([line_x[:, 1:], line_x[:, -1:]], axis=1)
    grid = R // BZ
    return pl.pallas_call(
        _build_volume_body,
        grid=(grid,),
        in_specs=[
            pl.BlockSpec((C, R), lambda g: (0, 0)),
            pl.BlockSpec((C, R), lambda g: (0, 0)),
            pl.BlockSpec((C, R), lambda g: (0, 0)),
            pl.BlockSpec((C, R), lambda g: (0, 0)),
        ],
        out_specs=pl.BlockSpec((BZ * 2 * R, 128), lambda g: (g, 0)),
        out_shape=jax.ShapeDtypeStruct((2 * R * R, 128), jnp.uint32),
    )(line_z, line_y, line_x, lx1)


def _sample_body(cx_hbm, cy_hbm, cz_hbm, vol_hbm, out_hbm, *scratch):
    bufs = (scratch[0:9], scratch[9:18])
    sems = scratch[18:20]
    wid = lax.axis_index("s") * 2 + lax.axis_index("c")
    base = wid * PTS_PER_WORKER

    def axis_prep(coord):
        pos = (coord + 1.0) * ((R - 1) * 0.5)
        i0 = jnp.clip(pos.astype(jnp.int32), 0, R - 2)
        w = pos - i0.astype(jnp.float32)
        return i0, w

    def unpack_lo(v):
        return plsc.bitcast(v << 16, jnp.float32)

    def unpack_hi(v):
        return plsc.bitcast(v & jnp.int32(-65536), jnp.float32)

    def stage_and_fire(k, buf, sem):
        cx_v, cy_v, cz_v, wx_v, wy_v, wz_v, idx_v, val_v, _ = buf
        off = base + k * CHUNK
        pltpu.sync_copy(cx_hbm.at[pl.ds(off, CHUNK)], cx_v)
        pltpu.sync_copy(cy_hbm.at[pl.ds(off, CHUNK)], cy_v)
        pltpu.sync_copy(cz_hbm.at[pl.ds(off, CHUNK)], cz_v)

        def pass_a(g, _):
            s = pl.ds(g * 16, 16)
            ix0, wx = axis_prep(cx_v[s])
            iy0, wy = axis_prep(cy_v[s])
            iz0, wz = axis_prep(cz_v[s])
            wx_v[s] = wx
            wy_v[s] = wy
            wz_v[s] = wz
            bw = (iz0 << 16) + ((ix0 >> 7) << 15) + (iy0 << 7) + (ix0 & 127)
            idx_v[pl.ds(0 * CHUNK + g * 16, 16)] = bw
            idx_v[pl.ds(1 * CHUNK + g * 16, 16)] = bw + 128
            idx_v[pl.ds(2 * CHUNK + g * 16, 16)] = bw + 65536
            idx_v[pl.ds(3 * CHUNK + g * 16, 16)] = bw + 65536 + 128
            return 0

        lax.fori_loop(0, GROUPS, pass_a, 0)
        return pltpu.async_copy(vol_hbm.at[idx_v], val_v, sem)

    def finish(k, buf, handle):
        _, _, _, wx_v, wy_v, wz_v, idx_v, val_v, out_v = buf
        off = base + k * CHUNK
        handle.wait()

        def pass_b(g, _):
            s = pl.ds(g * 16, 16)
            v00 = val_v[pl.ds(0 * CHUNK + g * 16, 16)]
            v01 = val_v[pl.ds(1 * CHUNK + g * 16, 16)]
            v10 = val_v[pl.ds(2 * CHUNK + g * 16, 16)]
            v11 = val_v[pl.ds(3 * CHUNK + g * 16, 16)]
            wx = wx_v[s]
            wy = wy_v[s]
            wz = wz_v[s]

            def xl(v):
                f0 = unpack_lo(v)
                f1 = unpack_hi(v)
                return f0 + wx * (f1 - f0)

            q00 = xl(v00)
            q01 = xl(v01)
            q10 = xl(v10)
            q11 = xl(v11)
            r0 = q00 + wy * (q01 - q00)
            r1 = q10 + wy * (q11 - q10)
            out_v[s] = r0 + wz * (r1 - r0)
            return 0

        lax.fori_loop(0, GROUPS, pass_b, 0)
        pltpu.sync_copy(out_v, out_hbm.at[pl.ds(off, CHUNK)])

    def do_pair(kk, _):
        k0 = kk * 2
        k1 = kk * 2 + 1
        h0 = stage_and_fire(k0, bufs[0], sems[0])
        h1 = stage_and_fire(k1, bufs[1], sems[1])
        finish(k0, bufs[0], h0)
        finish(k1, bufs[1], h1)
        return 0

    lax.fori_loop(0, NUM_CHUNKS // 2, do_pair, 0)


def _chunk_scratch():
    return [
        pltpu.VMEM((CHUNK,), jnp.float32),
        pltpu.VMEM((CHUNK,), jnp.float32),
        pltpu.VMEM((CHUNK,), jnp.float32),
        pltpu.VMEM((CHUNK,), jnp.float32),
        pltpu.VMEM((CHUNK,), jnp.float32),
        pltpu.VMEM((CHUNK,), jnp.float32),
        pltpu.VMEM((IDX_PER_CHUNK,), jnp.int32),
        pltpu.VMEM((IDX_PER_CHUNK,), jnp.int32),
        pltpu.VMEM((CHUNK,), jnp.float32),
    ]


@jax.jit
def kernel(in_tensor, line_z, line_y, line_x):
    vol = _build_volume(line_z, line_y, line_x).reshape(-1).view(jnp.int32)
    mesh = plsc.VectorSubcoreMesh(core_axis_name="c", subcore_axis_name="s")
    run = functools.partial(
        pl.kernel,
        mesh=mesh,
        compiler_params=pltpu.CompilerParams(needs_layout_passes=False),
        out_type=jax.ShapeDtypeStruct((N,), jnp.float32),
        scratch_types=_chunk_scratch() + _chunk_scratch() + [
            pltpu.SemaphoreType.DMA,
            pltpu.SemaphoreType.DMA,
        ],
    )(_sample_body)
    return run(in_tensor[:, 0], in_tensor[:, 1], in_tensor[:, 2], vol)


# speedup vs baseline: 51.8924x; 1.0009x over previous
import functools

import jax
import jax.numpy as jnp
from jax import lax
from jax.experimental import pallas as pl
from jax.experimental.pallas import tpu as pltpu, tpu_sc as plsc

C = 96
R = 256
N = 524288

BZ = 16

NUM_WORKERS = 32
PTS_PER_WORKER = N // NUM_WORKERS
CHUNK = 2048
NUM_CHUNKS = PTS_PER_WORKER // CHUNK
GROUPS = CHUNK // 16
IDX_PER_CHUNK = 4 * CHUNK


def _build_volume_body(lz_ref, ly_ref, lx0_ref, lx1_ref, out_ref):
    gstep = pl.program_id(0)
    lz = lz_ref[...]
    ly = ly_ref[...]
    lx0 = lx0_ref[...]
    lx1 = lx1_ref[...]
    lane = lax.broadcasted_iota(jnp.int32, (1, R), 1)
    for b in range(BZ):
        onehot = (lane == gstep * BZ + b).astype(jnp.float32)
        wcol = jnp.sum(lz * onehot, axis=1, keepdims=True)
        w = wcol * ly
        dn = (((0,), (0,)), ((), ()))
        g0 = lax.dot_general(w, lx0, dn, preferred_element_type=jnp.float32)
        g1 = lax.dot_general(w, lx1, dn, preferred_element_type=jnp.float32)
        u0 = lax.bitcast_convert_type(g0, jnp.uint32)
        u1 = lax.bitcast_convert_type(g1, jnp.uint32)
        word = ((u0 + 0x8000) >> 16) | ((u1 + 0x8000) & jnp.uint32(0xFFFF0000))
        out_ref[pl.ds((b * 2 + 0) * 256, 256), :] = word[:, 0:128]
        out_ref[pl.ds((b * 2 + 1) * 256, 256), :] = word[:, 128:256]


def _build_volume(line_z, line_y, line_x):
    lx1 = jnp.concatenate([line_x[:, 1:], line_x[:, -1:]], axis=1)
    grid = R // BZ
    return pl.pallas_call(
        _build_volume_body,
        grid=(grid,),
        in_specs=[
            pl.BlockSpec((C, R), lambda g: (0, 0)),
            pl.BlockSpec((C, R), lambda g: (0, 0)),
            pl.BlockSpec((C, R), lambda g: (0, 0)),
            pl.BlockSpec((C, R), lambda g: (0, 0)),
        ],
        out_specs=pl.BlockSpec((BZ * 2 * R, 128), lambda g: (g, 0)),
        out_shape=jax.ShapeDtypeStruct((2 * R * R, 128), jnp.uint32),
    )(line_z, line_y, line_x, lx1)


def _sample_body(cx_hbm, cy_hbm, cz_hbm, vol_hbm, out_hbm, *scratch):
    bufs = (scratch[0:9], scratch[9:18])
    sems = scratch[18:20]
    wid = lax.axis_index("s") * 2 + lax.axis_index("c")
    base = wid * PTS_PER_WORKER

    def axis_prep(coord):
        pos = (coord + 1.0) * ((R - 1) * 0.5)
        i0 = jnp.clip(pos.astype(jnp.int32), 0, R - 2)
        w = pos - i0.astype(jnp.float32)
        return i0, w

    def unpack_lo(v):
        return plsc.bitcast(v << 16, jnp.float32)

    def unpack_hi(v):
        return plsc.bitcast(v & jnp.int32(-65536), jnp.float32)

    def stage_and_fire(k, buf, sem):
        cx_v, cy_v, cz_v, wx_v, wy_v, wz_v, idx_v, val_v, _ = buf
        off = base + k * CHUNK
        pltpu.sync_copy(cx_hbm.at[pl.ds(off, CHUNK)], cx_v)
        pltpu.sync_copy(cy_hbm.at[pl.ds(off, CHUNK)], cy_v)
        pltpu.sync_copy(cz_hbm.at[pl.ds(off, CHUNK)], cz_v)

        def pass_a(g, _):
            s = pl.ds(g * 16, 16)
            ix0, wx = axis_prep(cx_v[s])
            iy0, wy = axis_prep(cy_v[s])
            iz0, wz = axis_prep(cz_v[s])
            wx_v[s] = wx
            wy_v[s] = wy
            wz_v[s] = wz
            bw = (iz0 << 16) + ((ix0 >> 7) << 15) + (iy0 << 7) + (ix0 & 127)
            idx_v[pl.ds(0 * CHUNK + g * 16, 16)] = bw
            idx_v[pl.ds(1 * CHUNK + g * 16, 16)] = bw + 128
            idx_v[pl.ds(2 * CHUNK + g * 16, 16)] = bw + 65536
            idx_v[pl.ds(3 * CHUNK + g * 16, 16)] = bw + 65536 + 128
            return 0

        lax.fori_loop(0, GROUPS, pass_a, 0)
        return pltpu.async_copy(vol_hbm.at[idx_v], val_v, sem)

    def finish(k, buf, handle):
        _, _, _, wx_v, wy_v, wz_v, idx_v, val_v, out_v = buf
        off = base + k * CHUNK
        handle.wait()

        def pass_b(g, _):
            s = pl.ds(g * 16, 16)
            v00 = val_v[pl.ds(0 * CHUNK + g * 16, 16)]
            v01 = val_v[pl.ds(1 * CHUNK + g * 16, 16)]
            v10 = val_v[pl.ds(2 * CHUNK + g * 16, 16)]
            v11 = val_v[pl.ds(3 * CHUNK + g * 16, 16)]
            wx = wx_v[s]
            wy = wy_v[s]
            wz = wz_v[s]

            def xl(v):
                f0 = unpack_lo(v)
                f1 = unpack_hi(v)
                return f0 + wx * (f1 - f0)

            q00 = xl(v00)
            q01 = xl(v01)
            q10 = xl(v10)
            q11 = xl(v11)
            r0 = q00 + wy * (q01 - q00)
            r1 = q10 + wy * (q11 - q10)
            out_v[s] = r0 + wz * (r1 - r0)
            return 0

        lax.fori_loop(0, GROUPS, pass_b, 0)
        pltpu.sync_copy(out_v, out_hbm.at[pl.ds(off, CHUNK)])

    def do_pair(kk, _):
        k0 = kk * 2
        k1 = kk * 2 + 1
        h0 = stage_and_fire(k0, bufs[0], sems[0])
        h1 = stage_and_fire(k1, bufs[1], sems[1])
        finish(k0, bufs[0], h0)
        finish(k1, bufs[1], h1)
        return 0

    lax.fori_loop(0, NUM_CHUNKS // 2, do_pair, 0)


def _chunk_scratch():
    return [
        pltpu.VMEM((CHUNK,), jnp.float32),
        pltpu.VMEM((CHUNK,), jnp.float32),
        pltpu.VMEM((CHUNK,), jnp.float32),
        pltpu.VMEM((CHUNK,), jnp.float32),
        pltpu.VMEM((CHUNK,), jnp.float32),
        pltpu.VMEM((CHUNK,), jnp.float32),
        pltpu.VMEM((IDX_PER_CHUNK,), jnp.int32),
        pltpu.VMEM((IDX_PER_CHUNK,), jnp.int32),
        pltpu.VMEM((CHUNK,), jnp.float32),
    ]


@jax.jit
def kernel(in_tensor, line_z, line_y, line_x):
    vol = _build_volume(line_z, line_y, line_x).reshape(-1).view(jnp.int32)
    mesh = plsc.VectorSubcoreMesh(core_axis_name="c", subcore_axis_name="s")
    run = functools.partial(
        pl.kernel,
        mesh=mesh,
        compiler_params=pltpu.CompilerParams(needs_layout_passes=False),
        out_type=jax.ShapeDtypeStruct((N,), jnp.float32),
        scratch_types=_chunk_scratch() + _chunk_scratch() + [
            pltpu.SemaphoreType.DMA,
            pltpu.SemaphoreType.DMA,
        ],
    )(_sample_body)
    return run(in_tensor[:, 0], in_tensor[:, 1], in_tensor[:, 2], vol)
